# baseline (speedup 1.0000x reference)
import functools

import jax
import jax.numpy as jnp
from jax import lax
from jax.experimental import pallas as pl
from jax.experimental.pallas import tpu as pltpu
from jax.experimental.pallas import tpu_sc as plsc

_EPS = 1e-5
_K = 96


def _make_sc_agg(n, e, d):
  info = plsc.get_sparse_core_info()
  nc, ns = info.num_cores, info.num_subcores
  nw = nc * ns
  ept = e // nw
  k = _K
  chunks = -(-ept // k)
  ept_pad = chunks * k
  rows_per_tile = n // ns
  zfull, zrem = divmod(rows_per_tile, k)
  assert e % nw == 0 and n % ns == 0 and zrem <= k
  mesh = plsc.VectorSubcoreMesh(core_axis_name="c", subcore_axis_name="s")

  @functools.partial(
      pl.kernel,
      out_type=jax.ShapeDtypeStruct((nc, n, d), jnp.float32),
      mesh=mesh,
      scratch_types=[
          pltpu.VMEM((ept_pad,), jnp.int32),
          pltpu.VMEM((chunks, k), jnp.int32),
          pltpu.VMEM((k, d), jnp.float32),
          pltpu.VMEM((k, d), jnp.float32),
          pltpu.VMEM_SHARED((n + 8, d), jnp.float32),
          pltpu.SemaphoreType.DMA,
          pltpu.SemaphoreType.DMA,
          pltpu.SemaphoreType.DMA,
          pltpu.SemaphoreType.DMA,
          pltpu.SemaphoreType.DMA,
      ],
  )
  def sc_agg(h_hbm, src_hbm, dst_hbm, out_hbm, src_v, dst_v, rows_a,
             rows_b, acc, sem_i, sem_a, sem_b, ssem_a, ssem_b):
    cid = lax.axis_index("c")
    sid = lax.axis_index("s")
    wid = cid * ns + sid

    idx_a = pltpu.async_copy(src_hbm.at[wid], src_v, sem_i)
    idx_b = pltpu.async_copy(dst_hbm.at[wid], dst_v, sem_i)

    zero16 = jnp.zeros((16,), jnp.float32)

    def zfill(r, _):
      for c in range(d // 16):
        rows_a[r, pl.ds(c * 16, 16)] = zero16
      return 0

    lax.fori_loop(0, k, zfill, 0)
    row0 = sid * rows_per_tile

    def zcopy(i, _):
      pltpu.async_copy(rows_a, acc.at[pl.ds(row0 + i * k, k)], sem_a)
      return 0

    lax.fori_loop(0, zfull, zcopy, 0)
    if zrem:
      pltpu.async_copy(rows_a.at[pl.ds(0, zrem)],
                       acc.at[pl.ds(row0 + zfull * k, zrem)], sem_a)

    def zdrain(i, _):
      pltpu.make_async_copy(rows_a, acc.at[pl.ds(row0 + i * k, k)],
                            sem_a).wait()
      return 0

    lax.fori_loop(0, zfull, zdrain, 0)
    if zrem:
      pltpu.make_async_copy(rows_a.at[pl.ds(0, zrem)],
                            acc.at[pl.ds(row0 + zfull * k, zrem)],
                            sem_a).wait()
    idx_a.wait()
    idx_b.wait()
    plsc.subcore_barrier()

    bufs = (rows_a, rows_b)
    gsems = (sem_a, sem_b)
    ssems = (ssem_a, ssem_b)
    pltpu.async_copy(h_hbm.at[src_v.at[pl.ds(0, k)]], rows_a, sem_a)

    def edge_pair(i0, _):
      for b in range(2):
        i = i0 + b
        rows = bufs[b]
        pltpu.make_async_copy(
            h_hbm.at[src_v.at[pl.ds(i * k, k)]], rows, gsems[b]).wait()
        pltpu.async_copy(rows, acc.at[dst_v.at[i]], ssems[b], add=True)

        @pl.when(i + 1 < chunks)
        def _():
          @pl.when(i >= 1)
          def _():
            pltpu.make_async_copy(
                bufs[1 - b], acc.at[dst_v.at[i]], ssems[1 - b]).wait()

          pltpu.async_copy(
              h_hbm.at[src_v.at[pl.ds((i + 1) * k, k)]], bufs[1 - b],
              gsems[1 - b])
      return 0

    lax.fori_loop(0, chunks // 2, lambda j, c: edge_pair(j * 2, c), 0)
    if chunks % 2:
      i = chunks - 1
      rows = bufs[i % 2]
      pltpu.make_async_copy(
          h_hbm.at[src_v.at[pl.ds(i * k, k)]], rows, gsems[i % 2]).wait()
      pltpu.async_copy(rows, acc.at[dst_v.at[i]], ssems[i % 2], add=True)
    pltpu.make_async_copy(bufs[0], acc.at[dst_v.at[0]], ssems[0]).wait()
    pltpu.make_async_copy(bufs[1], acc.at[dst_v.at[0]], ssems[1]).wait()
    plsc.subcore_barrier()

    out_rows = n // 10

    @pl.when(sid < 10)
    def _():
      pltpu.sync_copy(
          acc.at[pl.ds(sid * out_rows, out_rows)],
          out_hbm.at[cid, pl.ds(sid * out_rows, out_rows)],
      )

  return sc_agg


def _bn_relu(z, gamma, beta):
  m = jnp.mean(z, axis=0)
  zc = z - m
  v = jnp.mean(zc * zc, axis=0)
  return jnp.maximum(zc * lax.rsqrt(v + _EPS) * gamma + beta, 0.0)


def _mlp_and_readout(g, h, a, w1, b1, g1, be1, w2, b2, go, beo, bat):
  z = h[...] + a[0] + a[1]
  z = jnp.dot(z, w1[...], preferred_element_type=jnp.float32) + b1[...]
  z = _bn_relu(z, g1[...], be1[...])
  z = jnp.dot(z, w2[...], preferred_element_type=jnp.float32) + b2[...]
  z = _bn_relu(z, go[...], beo[...])
  n = z.shape[0]
  oh = (bat[...] == lax.broadcasted_iota(jnp.int32, (n, g), 1)).astype(
      jnp.float32)
  ro = lax.dot_general(
      oh, z, (((0,), (0,)), ((), ())), preferred_element_type=jnp.float32)
  return z, ro


def _tc_layer_body(g, h, a, w1, b1, g1, be1, w2, b2, go, beo, bat, hout, rout):
  z, ro = _mlp_and_readout(g, h, a, w1, b1, g1, be1, w2, b2, go, beo, bat)
  hout[...] = z
  rout[...] = ro


def _tc_last_body(g, hdim, h, a, w1, b1, g1, be1, w2, b2, go, beo, bat,
                  r0, r1, wc1, bc1, wc2, bc2, out):
  _, ro2 = _mlp_and_readout(g, h, a, w1, b1, g1, be1, w2, b2, go, beo, bat)
  hg = (jnp.dot(r0[...], wc1[pl.ds(0, hdim)],
                preferred_element_type=jnp.float32) +
        jnp.dot(r1[...], wc1[pl.ds(hdim, hdim)],
                preferred_element_type=jnp.float32) +
        jnp.dot(ro2, wc1[pl.ds(2 * hdim, hdim)],
                preferred_element_type=jnp.float32) + bc1[...])
  hc = jnp.maximum(hg, 0.0)
  out[...] = jnp.dot(hc, wc2[...],
                     preferred_element_type=jnp.float32) + bc2[...]


def _layer_args(p, bat2d):
  return (p["W1"], p["b1"][None, :], p["g1"][None, :], p["be1"][None, :],
          p["W2"], p["b2"][None, :], p["g_out"][None, :], p["be_out"][None, :],
          bat2d)


def kernel(x, params, edge_index, batch):
  n, d = x.shape
  e = edge_index.shape[1]
  num_layers = sum(1 for key in params if key.startswith("layer"))
  g = 64

  sc_agg = _make_sc_agg(n, e, d)
  nw = 32
  ept = e // nw
  chunks = -(-ept // _K)
  pad = chunks * _K - ept
  src = jnp.concatenate(
      [edge_index[0].reshape(nw, ept),
       jnp.zeros((nw, pad), jnp.int32)], axis=1)
  dst = jnp.concatenate(
      [edge_index[1].reshape(nw, ept),
       jnp.full((nw, pad), n, jnp.int32)], axis=1).reshape(nw, chunks, _K)
  bat2d = batch[:, None]

  h = x
  readouts = []
  for i in range(num_layers - 1):
    p = params["layer%d" % i]
    aggs = sc_agg(h, src, dst)
    h, ro = pl.pallas_call(
        functools.partial(_tc_layer_body, g),
        out_shape=(
            jax.ShapeDtypeStruct((n, p["W2"].shape[1]), jnp.float32),
            jax.ShapeDtypeStruct((g, p["W2"].shape[1]), jnp.float32),
        ),
    )(h, aggs, *_layer_args(p, bat2d))
    readouts.append(ro)

  p = params["layer%d" % (num_layers - 1)]
  hdim = params["Wc2"].shape[0]
  c = params["Wc2"].shape[1]
  aggs = sc_agg(h, src, dst)
  out = pl.pallas_call(
      functools.partial(_tc_last_body, g, hdim),
      out_shape=jax.ShapeDtypeStruct((g, c), jnp.float32),
  )(h, aggs, *_layer_args(p, bat2d), readouts[0], readouts[1],
    params["Wc1"], params["bc1"][None, :], params["Wc2"],
    params["bc2"][None, :])
  return out


# speedup vs baseline: 1.0013x; 1.0013x over previous
import functools

import jax
import jax.numpy as jnp
from jax import lax
from jax.experimental import pallas as pl
from jax.experimental.pallas import tpu as pltpu
from jax.experimental.pallas import tpu_sc as plsc

_EPS = 1e-5
_K = 96


def _make_sc_agg(n, e, d):
  info = plsc.get_sparse_core_info()
  nc, ns = info.num_cores, info.num_subcores
  nw = nc * ns
  ept = e // nw
  k = _K
  chunks = -(-ept // k)
  ept_pad = chunks * k
  rows_per_tile = n // ns
  zfull, zrem = divmod(rows_per_tile, k)
  assert e % nw == 0 and n % ns == 0 and zrem <= k
  mesh = plsc.VectorSubcoreMesh(core_axis_name="c", subcore_axis_name="s")

  @functools.partial(
      pl.kernel,
      out_type=jax.ShapeDtypeStruct((nc, n, d), jnp.float32),
      mesh=mesh,
      scratch_types=[
          pltpu.VMEM((ept_pad,), jnp.int32),
          pltpu.VMEM((chunks, k), jnp.int32),
          pltpu.VMEM((k, d), jnp.float32),
          pltpu.VMEM((k, d), jnp.float32),
          pltpu.VMEM_SHARED((n + ns, d), jnp.float32),
          pltpu.SemaphoreType.DMA,
          pltpu.SemaphoreType.DMA,
          pltpu.SemaphoreType.DMA,
          pltpu.SemaphoreType.DMA,
          pltpu.SemaphoreType.DMA,
      ],
  )
  def sc_agg(h_hbm, src_hbm, dst_hbm, out_hbm, src_v, dst_v, rows_a,
             rows_b, acc, sem_i, sem_a, sem_b, ssem_a, ssem_b):
    cid = lax.axis_index("c")
    sid = lax.axis_index("s")
    wid = cid * ns + sid

    idx_a = pltpu.async_copy(src_hbm.at[wid], src_v, sem_i)
    idx_b = pltpu.async_copy(dst_hbm.at[wid], dst_v, sem_i)

    zero16 = jnp.zeros((16,), jnp.float32)

    def zfill(r, _):
      for c in range(d // 16):
        rows_a[r, pl.ds(c * 16, 16)] = zero16
      return 0

    lax.fori_loop(0, k, zfill, 0)
    row0 = sid * rows_per_tile

    def zcopy(i, _):
      pltpu.async_copy(rows_a, acc.at[pl.ds(row0 + i * k, k)], sem_a)
      return 0

    lax.fori_loop(0, zfull, zcopy, 0)
    if zrem:
      pltpu.async_copy(rows_a.at[pl.ds(0, zrem)],
                       acc.at[pl.ds(row0 + zfull * k, zrem)], sem_a)

    def zdrain(i, _):
      pltpu.make_async_copy(rows_a, acc.at[pl.ds(row0 + i * k, k)],
                            sem_a).wait()
      return 0

    lax.fori_loop(0, zfull, zdrain, 0)
    if zrem:
      pltpu.make_async_copy(rows_a.at[pl.ds(0, zrem)],
                            acc.at[pl.ds(row0 + zfull * k, zrem)],
                            sem_a).wait()
    idx_a.wait()
    idx_b.wait()
    plsc.subcore_barrier()

    bufs = (rows_a, rows_b)
    gsems = (sem_a, sem_b)
    ssems = (ssem_a, ssem_b)
    pltpu.async_copy(h_hbm.at[src_v.at[pl.ds(0, k)]], rows_a, sem_a)

    def edge_pair(i0, _):
      for b in range(2):
        i = i0 + b
        rows = bufs[b]
        pltpu.make_async_copy(
            h_hbm.at[src_v.at[pl.ds(i * k, k)]], rows, gsems[b]).wait()
        pltpu.async_copy(rows, acc.at[dst_v.at[i]], ssems[b], add=True)

        @pl.when(i + 1 < chunks)
        def _():
          @pl.when(i >= 1)
          def _():
            pltpu.make_async_copy(
                bufs[1 - b], acc.at[dst_v.at[i]], ssems[1 - b]).wait()

          pltpu.async_copy(
              h_hbm.at[src_v.at[pl.ds((i + 1) * k, k)]], bufs[1 - b],
              gsems[1 - b])
      return 0

    lax.fori_loop(0, chunks // 2, lambda j, c: edge_pair(j * 2, c), 0)
    if chunks % 2:
      i = chunks - 1
      rows = bufs[i % 2]
      pltpu.make_async_copy(
          h_hbm.at[src_v.at[pl.ds(i * k, k)]], rows, gsems[i % 2]).wait()
      pltpu.async_copy(rows, acc.at[dst_v.at[i]], ssems[i % 2], add=True)
    pltpu.make_async_copy(bufs[0], acc.at[dst_v.at[0]], ssems[0]).wait()
    pltpu.make_async_copy(bufs[1], acc.at[dst_v.at[0]], ssems[1]).wait()
    plsc.subcore_barrier()

    out_rows = n // 10

    @pl.when(sid < 10)
    def _():
      pltpu.sync_copy(
          acc.at[pl.ds(sid * out_rows, out_rows)],
          out_hbm.at[cid, pl.ds(sid * out_rows, out_rows)],
      )

  return sc_agg


def _bn_relu(z, gamma, beta):
  m = jnp.mean(z, axis=0)
  zc = z - m
  v = jnp.mean(zc * zc, axis=0)
  return jnp.maximum(zc * lax.rsqrt(v + _EPS) * gamma + beta, 0.0)


def _mlp_and_readout(g, h, a, w1, b1, g1, be1, w2, b2, go, beo, bat):
  z = h[...] + a[0] + a[1]
  z = jnp.dot(z, w1[...], preferred_element_type=jnp.float32) + b1[...]
  z = _bn_relu(z, g1[...], be1[...])
  z = jnp.dot(z, w2[...], preferred_element_type=jnp.float32) + b2[...]
  z = _bn_relu(z, go[...], beo[...])
  n = z.shape[0]
  oh = (bat[...] == lax.broadcasted_iota(jnp.int32, (n, g), 1)).astype(
      jnp.float32)
  ro = lax.dot_general(
      oh, z, (((0,), (0,)), ((), ())), preferred_element_type=jnp.float32)
  return z, ro


def _tc_layer_body(g, h, a, w1, b1, g1, be1, w2, b2, go, beo, bat, hout, rout):
  z, ro = _mlp_and_readout(g, h, a, w1, b1, g1, be1, w2, b2, go, beo, bat)
  hout[...] = z
  rout[...] = ro


def _tc_last_body(g, hdim, h, a, w1, b1, g1, be1, w2, b2, go, beo, bat,
                  r0, r1, wc1, bc1, wc2, bc2, out):
  _, ro2 = _mlp_and_readout(g, h, a, w1, b1, g1, be1, w2, b2, go, beo, bat)
  hg = (jnp.dot(r0[...], wc1[pl.ds(0, hdim)],
                preferred_element_type=jnp.float32) +
        jnp.dot(r1[...], wc1[pl.ds(hdim, hdim)],
                preferred_element_type=jnp.float32) +
        jnp.dot(ro2, wc1[pl.ds(2 * hdim, hdim)],
                preferred_element_type=jnp.float32) + bc1[...])
  hc = jnp.maximum(hg, 0.0)
  out[...] = jnp.dot(hc, wc2[...],
                     preferred_element_type=jnp.float32) + bc2[...]


def _layer_args(p, bat2d):
  return (p["W1"], p["b1"][None, :], p["g1"][None, :], p["be1"][None, :],
          p["W2"], p["b2"][None, :], p["g_out"][None, :], p["be_out"][None, :],
          bat2d)


def kernel(x, params, edge_index, batch):
  n, d = x.shape
  e = edge_index.shape[1]
  num_layers = sum(1 for key in params if key.startswith("layer"))
  g = 64

  sc_agg = _make_sc_agg(n, e, d)
  nw = 32
  ept = e // nw
  chunks = -(-ept // _K)
  pad = chunks * _K - ept
  dummy_row = n + (jnp.arange(nw, dtype=jnp.int32) % 16)[:, None]
  src = jnp.concatenate(
      [edge_index[0].reshape(nw, ept),
       jnp.zeros((nw, pad), jnp.int32)], axis=1)
  dst = jnp.concatenate(
      [edge_index[1].reshape(nw, ept),
       jnp.broadcast_to(dummy_row, (nw, pad))], axis=1).reshape(
           nw, chunks, _K)
  bat2d = batch[:, None]

  h = x
  readouts = []
  for i in range(num_layers - 1):
    p = params["layer%d" % i]
    aggs = sc_agg(h, src, dst)
    h, ro = pl.pallas_call(
        functools.partial(_tc_layer_body, g),
        out_shape=(
            jax.ShapeDtypeStruct((n, p["W2"].shape[1]), jnp.float32),
            jax.ShapeDtypeStruct((g, p["W2"].shape[1]), jnp.float32),
        ),
    )(h, aggs, *_layer_args(p, bat2d))
    readouts.append(ro)

  p = params["layer%d" % (num_layers - 1)]
  hdim = params["Wc2"].shape[0]
  c = params["Wc2"].shape[1]
  aggs = sc_agg(h, src, dst)
  out = pl.pallas_call(
      functools.partial(_tc_last_body, g, hdim),
      out_shape=jax.ShapeDtypeStruct((g, c), jnp.float32),
  )(h, aggs, *_layer_args(p, bat2d), readouts[0], readouts[1],
    params["Wc1"], params["bc1"][None, :], params["Wc2"],
    params["bc2"][None, :])
  return out


# speedup vs baseline: 1.4311x; 1.4293x over previous
import functools

import jax
import jax.numpy as jnp
from jax import lax
from jax.experimental import pallas as pl
from jax.experimental.pallas import tpu as pltpu
from jax.experimental.pallas import tpu_sc as plsc

_EPS = 1e-5
_K = 80


def _make_sc_agg(n, e, d):
  info = plsc.get_sparse_core_info()
  nc, ns = info.num_cores, info.num_subcores
  nw = nc * ns
  ept = e // nw
  k = _K
  chunks = -(-ept // k)
  ept_pad = chunks * k
  rows_per_tile = n // ns
  zfull, zrem = divmod(rows_per_tile, k)
  assert e % nw == 0 and n % ns == 0 and zrem <= k
  mesh = plsc.VectorSubcoreMesh(core_axis_name="c", subcore_axis_name="s")

  @functools.partial(
      pl.kernel,
      out_type=jax.ShapeDtypeStruct((nc, n, d), jnp.float32),
      mesh=mesh,
      scratch_types=[
          pltpu.VMEM((ept_pad,), jnp.int32),
          pltpu.VMEM((chunks, k), jnp.int32),
          pltpu.VMEM((k, d), jnp.float32),
          pltpu.VMEM((k, d), jnp.float32),
          pltpu.VMEM_SHARED((n + (ns if ept % _K else 0), d),
                            jnp.float32),
          pltpu.SemaphoreType.DMA,
          pltpu.SemaphoreType.DMA,
          pltpu.SemaphoreType.DMA,
          pltpu.SemaphoreType.DMA,
          pltpu.SemaphoreType.DMA,
      ],
  )
  def sc_agg(h_hbm, src_hbm, dst_hbm, out_hbm, src_v, dst_v, rows_a,
             rows_b, acc, sem_i, sem_a, sem_b, ssem_a, ssem_b):
    cid = lax.axis_index("c")
    sid = lax.axis_index("s")
    wid = cid * ns + sid

    idx_a = pltpu.async_copy(src_hbm.at[wid], src_v, sem_i)
    idx_b = pltpu.async_copy(dst_hbm.at[wid], dst_v, sem_i)

    zero16 = jnp.zeros((16,), jnp.float32)

    def zfill(r, _):
      for c in range(d // 16):
        rows_a[r, pl.ds(c * 16, 16)] = zero16
      return 0

    lax.fori_loop(0, k, zfill, 0)
    row0 = sid * rows_per_tile

    def zcopy(i, _):
      pltpu.async_copy(rows_a, acc.at[pl.ds(row0 + i * k, k)], sem_a)
      return 0

    lax.fori_loop(0, zfull, zcopy, 0)
    if zrem:
      pltpu.async_copy(rows_a.at[pl.ds(0, zrem)],
                       acc.at[pl.ds(row0 + zfull * k, zrem)], sem_a)

    def zdrain(i, _):
      pltpu.make_async_copy(rows_a, acc.at[pl.ds(row0 + i * k, k)],
                            sem_a).wait()
      return 0

    lax.fori_loop(0, zfull, zdrain, 0)
    if zrem:
      pltpu.make_async_copy(rows_a.at[pl.ds(0, zrem)],
                            acc.at[pl.ds(row0 + zfull * k, zrem)],
                            sem_a).wait()
    idx_a.wait()
    idx_b.wait()
    plsc.subcore_barrier()

    bufs = (rows_a, rows_b)
    gsems = (sem_a, sem_b)
    ssems = (ssem_a, ssem_b)
    pltpu.async_copy(h_hbm.at[src_v.at[pl.ds(0, k)]], rows_a, sem_a)

    def edge_pair(i0, _):
      for b in range(2):
        i = i0 + b
        rows = bufs[b]
        pltpu.make_async_copy(
            h_hbm.at[src_v.at[pl.ds(i * k, k)]], rows, gsems[b]).wait()
        pltpu.async_copy(rows, acc.at[dst_v.at[i]], ssems[b], add=True)

        @pl.when(i + 1 < chunks)
        def _():
          @pl.when(i >= 1)
          def _():
            pltpu.make_async_copy(
                bufs[1 - b], acc.at[dst_v.at[i]], ssems[1 - b]).wait()

          pltpu.async_copy(
              h_hbm.at[src_v.at[pl.ds((i + 1) * k, k)]], bufs[1 - b],
              gsems[1 - b])
      return 0

    lax.fori_loop(0, chunks // 2, lambda j, c: edge_pair(j * 2, c), 0)
    if chunks % 2:
      i = chunks - 1
      rows = bufs[i % 2]
      pltpu.make_async_copy(
          h_hbm.at[src_v.at[pl.ds(i * k, k)]], rows, gsems[i % 2]).wait()
      pltpu.async_copy(rows, acc.at[dst_v.at[i]], ssems[i % 2], add=True)
    pltpu.make_async_copy(bufs[0], acc.at[dst_v.at[0]], ssems[0]).wait()
    pltpu.make_async_copy(bufs[1], acc.at[dst_v.at[0]], ssems[1]).wait()
    plsc.subcore_barrier()

    out_rows = n // 10

    @pl.when(sid < 10)
    def _():
      pltpu.sync_copy(
          acc.at[pl.ds(sid * out_rows, out_rows)],
          out_hbm.at[cid, pl.ds(sid * out_rows, out_rows)],
      )

  return sc_agg


def _bn_relu(z, gamma, beta):
  m = jnp.mean(z, axis=0)
  zc = z - m
  v = jnp.mean(zc * zc, axis=0)
  return jnp.maximum(zc * lax.rsqrt(v + _EPS) * gamma + beta, 0.0)


def _mlp_and_readout(g, h, a, w1, b1, g1, be1, w2, b2, go, beo, bat):
  z = h[...] + a[0] + a[1]
  z = jnp.dot(z, w1[...], preferred_element_type=jnp.float32) + b1[...]
  z = _bn_relu(z, g1[...], be1[...])
  z = jnp.dot(z, w2[...], preferred_element_type=jnp.float32) + b2[...]
  z = _bn_relu(z, go[...], beo[...])
  n = z.shape[0]
  oh = (bat[...] == lax.broadcasted_iota(jnp.int32, (n, g), 1)).astype(
      jnp.float32)
  ro = lax.dot_general(
      oh, z, (((0,), (0,)), ((), ())), preferred_element_type=jnp.float32)
  return z, ro


def _tc_layer_body(g, h, a, w1, b1, g1, be1, w2, b2, go, beo, bat, hout, rout):
  z, ro = _mlp_and_readout(g, h, a, w1, b1, g1, be1, w2, b2, go, beo, bat)
  hout[...] = z
  rout[...] = ro


def _tc_last_body(g, hdim, h, a, w1, b1, g1, be1, w2, b2, go, beo, bat,
                  r0, r1, wc1, bc1, wc2, bc2, out):
  _, ro2 = _mlp_and_readout(g, h, a, w1, b1, g1, be1, w2, b2, go, beo, bat)
  hg = (jnp.dot(r0[...], wc1[pl.ds(0, hdim)],
                preferred_element_type=jnp.float32) +
        jnp.dot(r1[...], wc1[pl.ds(hdim, hdim)],
                preferred_element_type=jnp.float32) +
        jnp.dot(ro2, wc1[pl.ds(2 * hdim, hdim)],
                preferred_element_type=jnp.float32) + bc1[...])
  hc = jnp.maximum(hg, 0.0)
  out[...] = jnp.dot(hc, wc2[...],
                     preferred_element_type=jnp.float32) + bc2[...]


def _layer_args(p, bat2d):
  return (p["W1"], p["b1"][None, :], p["g1"][None, :], p["be1"][None, :],
          p["W2"], p["b2"][None, :], p["g_out"][None, :], p["be_out"][None, :],
          bat2d)


def kernel(x, params, edge_index, batch):
  n, d = x.shape
  e = edge_index.shape[1]
  num_layers = sum(1 for key in params if key.startswith("layer"))
  g = 64

  sc_agg = _make_sc_agg(n, e, d)
  nw = 32
  ept = e // nw
  chunks = -(-ept // _K)
  pad = chunks * _K - ept
  if pad:
    dummy_row = n + (jnp.arange(nw, dtype=jnp.int32) % 16)[:, None]
    src = jnp.concatenate(
        [edge_index[0].reshape(nw, ept),
         jnp.zeros((nw, pad), jnp.int32)], axis=1)
    dst = jnp.concatenate(
        [edge_index[1].reshape(nw, ept),
         jnp.broadcast_to(dummy_row, (nw, pad))], axis=1).reshape(
             nw, chunks, _K)
  else:
    src = edge_index[0].reshape(nw, ept)
    dst = edge_index[1].reshape(nw, chunks, _K)
  bat2d = batch[:, None]

  h = x
  readouts = []
  for i in range(num_layers - 1):
    p = params["layer%d" % i]
    aggs = sc_agg(h, src, dst)
    h, ro = pl.pallas_call(
        functools.partial(_tc_layer_body, g),
        out_shape=(
            jax.ShapeDtypeStruct((n, p["W2"].shape[1]), jnp.float32),
            jax.ShapeDtypeStruct((g, p["W2"].shape[1]), jnp.float32),
        ),
    )(h, aggs, *_layer_args(p, bat2d))
    readouts.append(ro)

  p = params["layer%d" % (num_layers - 1)]
  hdim = params["Wc2"].shape[0]
  c = params["Wc2"].shape[1]
  aggs = sc_agg(h, src, dst)
  out = pl.pallas_call(
      functools.partial(_tc_last_body, g, hdim),
      out_shape=jax.ShapeDtypeStruct((g, c), jnp.float32),
  )(h, aggs, *_layer_args(p, bat2d), readouts[0], readouts[1],
    params["Wc1"], params["bc1"][None, :], params["Wc2"],
    params["bc2"][None, :])
  return out


# speedup vs baseline: 1.4388x; 1.0054x over previous
import functools

import jax
import jax.numpy as jnp
from jax import lax
from jax.experimental import pallas as pl
from jax.experimental.pallas import tpu as pltpu
from jax.experimental.pallas import tpu_sc as plsc

_EPS = 1e-5
_K = 80


def _make_sc_agg(n, e, d):
  info = plsc.get_sparse_core_info()
  nc, ns = info.num_cores, info.num_subcores
  nw = nc * ns
  ept = e // nw
  k = _K
  chunks = -(-ept // k)
  ept_pad = chunks * k
  rows_per_tile = n // ns
  zfull, zrem = divmod(rows_per_tile, k)
  assert e % nw == 0 and n % ns == 0 and zrem <= k
  mesh = plsc.VectorSubcoreMesh(core_axis_name="c", subcore_axis_name="s")

  @functools.partial(
      pl.kernel,
      out_type=jax.ShapeDtypeStruct((nc, n, d), jnp.float32),
      mesh=mesh,
      scratch_types=[
          pltpu.VMEM((ept_pad,), jnp.int32),
          pltpu.VMEM((chunks, k), jnp.int32),
          pltpu.VMEM((k, d), jnp.float32),
          pltpu.VMEM((k, d), jnp.float32),
          pltpu.VMEM_SHARED((n + (ns if ept % _K else 0), d),
                            jnp.float32),
          pltpu.SemaphoreType.DMA,
          pltpu.SemaphoreType.DMA,
          pltpu.SemaphoreType.DMA,
      ],
  )
  def sc_agg(h_hbm, src_hbm, dst_hbm, out_hbm, src_v, dst_v, rows_a,
             rows_b, acc, sem_i, sem_a, sem_b):
    cid = lax.axis_index("c")
    sid = lax.axis_index("s")
    wid = cid * ns + sid

    idx_a = pltpu.async_copy(src_hbm.at[wid], src_v, sem_i)
    idx_b = pltpu.async_copy(dst_hbm.at[wid], dst_v, sem_i)

    zero16 = jnp.zeros((16,), jnp.float32)

    def zfill(r, _):
      for c in range(d // 16):
        rows_a[r, pl.ds(c * 16, 16)] = zero16
      return 0

    lax.fori_loop(0, k, zfill, 0)
    row0 = sid * rows_per_tile

    def zcopy(i, _):
      pltpu.sync_copy(rows_a, acc.at[pl.ds(row0 + i * k, k)])
      return 0

    lax.fori_loop(0, zfull, zcopy, 0)
    if zrem:
      pltpu.sync_copy(rows_a.at[pl.ds(0, zrem)],
                      acc.at[pl.ds(row0 + zfull * k, zrem)])
    idx_a.wait()
    idx_b.wait()
    plsc.subcore_barrier()

    bufs = (rows_a, rows_b)
    gsems = (sem_a, sem_b)
    pltpu.async_copy(h_hbm.at[src_v.at[pl.ds(0, k)]], rows_a, sem_a)

    def edge_pair(i0, _):
      for b in range(2):
        i = i0 + b
        rows = bufs[b]
        pltpu.make_async_copy(
            h_hbm.at[src_v.at[pl.ds(i * k, k)]], rows, gsems[b]).wait()

        @pl.when(i + 1 < chunks)
        def _():
          pltpu.async_copy(
              h_hbm.at[src_v.at[pl.ds((i + 1) * k, k)]], bufs[1 - b],
              gsems[1 - b])

        pltpu.sync_copy(rows, acc.at[dst_v.at[i]], add=True)
      return 0

    lax.fori_loop(0, chunks // 2, lambda j, c: edge_pair(j * 2, c), 0)
    if chunks % 2:
      i = chunks - 1
      rows = bufs[i % 2]
      pltpu.make_async_copy(
          h_hbm.at[src_v.at[pl.ds(i * k, k)]], rows, gsems[i % 2]).wait()
      pltpu.sync_copy(rows, acc.at[dst_v.at[i]], add=True)
    plsc.subcore_barrier()

    out_rows = n // 10

    @pl.when(sid < 10)
    def _():
      pltpu.sync_copy(
          acc.at[pl.ds(sid * out_rows, out_rows)],
          out_hbm.at[cid, pl.ds(sid * out_rows, out_rows)],
      )

  return sc_agg


def _bn_relu(z, gamma, beta):
  m = jnp.mean(z, axis=0)
  zc = z - m
  v = jnp.mean(zc * zc, axis=0)
  return jnp.maximum(zc * lax.rsqrt(v + _EPS) * gamma + beta, 0.0)


def _mlp_and_readout(g, h, a, w1, b1, g1, be1, w2, b2, go, beo, bat):
  z = h[...] + a[0] + a[1]
  z = jnp.dot(z, w1[...], preferred_element_type=jnp.float32) + b1[...]
  z = _bn_relu(z, g1[...], be1[...])
  z = jnp.dot(z, w2[...], preferred_element_type=jnp.float32) + b2[...]
  z = _bn_relu(z, go[...], beo[...])
  n = z.shape[0]
  oh = (bat[...] == lax.broadcasted_iota(jnp.int32, (n, g), 1)).astype(
      jnp.float32)
  ro = lax.dot_general(
      oh, z, (((0,), (0,)), ((), ())), preferred_element_type=jnp.float32)
  return z, ro


def _tc_layer_body(g, h, a, w1, b1, g1, be1, w2, b2, go, beo, bat, hout, rout):
  z, ro = _mlp_and_readout(g, h, a, w1, b1, g1, be1, w2, b2, go, beo, bat)
  hout[...] = z
  rout[...] = ro


def _tc_last_body(g, hdim, h, a, w1, b1, g1, be1, w2, b2, go, beo, bat,
                  r0, r1, wc1, bc1, wc2, bc2, out):
  _, ro2 = _mlp_and_readout(g, h, a, w1, b1, g1, be1, w2, b2, go, beo, bat)
  hg = (jnp.dot(r0[...], wc1[pl.ds(0, hdim)],
                preferred_element_type=jnp.float32) +
        jnp.dot(r1[...], wc1[pl.ds(hdim, hdim)],
                preferred_element_type=jnp.float32) +
        jnp.dot(ro2, wc1[pl.ds(2 * hdim, hdim)],
                preferred_element_type=jnp.float32) + bc1[...])
  hc = jnp.maximum(hg, 0.0)
  out[...] = jnp.dot(hc, wc2[...],
                     preferred_element_type=jnp.float32) + bc2[...]


def _layer_args(p, bat2d):
  return (p["W1"], p["b1"][None, :], p["g1"][None, :], p["be1"][None, :],
          p["W2"], p["b2"][None, :], p["g_out"][None, :], p["be_out"][None, :],
          bat2d)


def kernel(x, params, edge_index, batch):
  n, d = x.shape
  e = edge_index.shape[1]
  num_layers = sum(1 for key in params if key.startswith("layer"))
  g = 64

  sc_agg = _make_sc_agg(n, e, d)
  nw = 32
  ept = e // nw
  chunks = -(-ept // _K)
  pad = chunks * _K - ept
  if pad:
    dummy_row = n + (jnp.arange(nw, dtype=jnp.int32) % 16)[:, None]
    src = jnp.concatenate(
        [edge_index[0].reshape(nw, ept),
         jnp.zeros((nw, pad), jnp.int32)], axis=1)
    dst = jnp.concatenate(
        [edge_index[1].reshape(nw, ept),
         jnp.broadcast_to(dummy_row, (nw, pad))], axis=1).reshape(
             nw, chunks, _K)
  else:
    src = edge_index[0].reshape(nw, ept)
    dst = edge_index[1].reshape(nw, chunks, _K)
  bat2d = batch[:, None]

  h = x
  readouts = []
  for i in range(num_layers - 1):
    p = params["layer%d" % i]
    aggs = sc_agg(h, src, dst)
    h, ro = pl.pallas_call(
        functools.partial(_tc_layer_body, g),
        out_shape=(
            jax.ShapeDtypeStruct((n, p["W2"].shape[1]), jnp.float32),
            jax.ShapeDtypeStruct((g, p["W2"].shape[1]), jnp.float32),
        ),
    )(h, aggs, *_layer_args(p, bat2d))
    readouts.append(ro)

  p = params["layer%d" % (num_layers - 1)]
  hdim = params["Wc2"].shape[0]
  c = params["Wc2"].shape[1]
  aggs = sc_agg(h, src, dst)
  out = pl.pallas_call(
      functools.partial(_tc_last_body, g, hdim),
      out_shape=jax.ShapeDtypeStruct((g, c), jnp.float32),
  )(h, aggs, *_layer_args(p, bat2d), readouts[0], readouts[1],
    params["Wc1"], params["bc1"][None, :], params["Wc2"],
    params["bc2"][None, :])
  return out
